# baseline (speedup 1.0000x reference)
import jax
import jax.numpy as jnp
from jax import lax
from jax.experimental import pallas as pl
from jax.experimental.pallas import tpu as pltpu
from jax.experimental.pallas import tpu_sc as plsc

_GAMMA = 12.0
_N_EDGES = 320000
_N_NODES = 10000
_D = 128
_DW = _D // 2
_NW = 32
_EPW = _N_EDGES // _NW
_CHUNK = 80
_NCHUNK = _EPW // _CHUNK
_NPAIR = (_NCHUNK - 1) // 2
_G16 = _CHUNK // 16


def _sc_body(node_hbm, src_hbm, dst_hbm, rel_hbm, out_hbm,
             src_all, dst_all,
             head0, tail0, rel0, score0, head1, tail1, rel1, score1,
             acc_buf, sem0, sem1, sems0, sems1):
    wid = lax.axis_index("s") * 2 + lax.axis_index("c")
    wbase = wid * _EPW
    lanes = lax.iota(jnp.int32, 16)

    pltpu.sync_copy(src_hbm.at[wid], src_all)
    pltpu.sync_copy(dst_hbm.at[wid], dst_all)

    def start(ci, head_v, tail_v, rel_v, sem):
        off = ci * _CHUNK
        pltpu.async_copy(rel_hbm.at[pl.ds(wbase + off, _CHUNK)], rel_v, sem)
        pltpu.async_copy(node_hbm.at[src_all.at[ci]], head_v, sem)
        pltpu.async_copy(node_hbm.at[dst_all.at[ci]], tail_v, sem)

    def wait_chunk(ci, head_v, tail_v, rel_v, sem):
        off = ci * _CHUNK
        pltpu.make_async_copy(node_hbm.at[src_all.at[ci]], head_v, sem).wait()
        pltpu.make_async_copy(node_hbm.at[dst_all.at[ci]], tail_v, sem).wait()
        pltpu.make_async_copy(rel_hbm.at[pl.ds(wbase + off, _CHUNK)],
                              rel_v, sem).wait()

    def compute(ci, head_v, tail_v, rel_v, score_v, sems):
        off = ci * _CHUNK

        @pl.when(ci >= 2)
        def _drain_prev():
            pltpu.make_async_copy(
                score_v, out_hbm.at[pl.ds(wbase + (ci - 2) * _CHUNK, _CHUNK)],
                sems).wait()

        def group_body(g, carry2):
            for e in range(16):
                row = g * 16 + e
                acc = jnp.zeros((16,), jnp.float32)
                for j in range(4):
                    hw = head_v[row, pl.ds(j * 16, 16)]
                    tw = tail_v[row, pl.ds(j * 16, 16)]
                    ha, hb = plsc.unpack(plsc.bitcast(hw, jnp.bfloat16),
                                         format=plsc.PackFormat.INTERLEAVED)
                    ta, tb = plsc.unpack(plsc.bitcast(tw, jnp.bfloat16),
                                         format=plsc.PackFormat.INTERLEAVED)
                    r0 = rel_v[row, pl.ds(j * 32, 16)]
                    r1 = rel_v[row, pl.ds(j * 32 + 16, 16)]
                    acc = acc + jnp.abs(ha + r0 - ta) + jnp.abs(hb + r1 - tb)
                acc_buf[e, pl.ds(0, 16)] = acc
            s = jnp.zeros((16,), jnp.float32)
            for l in range(16):
                col = jnp.full((16,), l, jnp.int32)
                s = s + plsc.load_gather(acc_buf, [lanes, col])
            score_v[pl.ds(g * 16, 16)] = _GAMMA - s
            return carry2

        lax.fori_loop(0, _G16, group_body, 0)
        pltpu.async_copy(score_v, out_hbm.at[pl.ds(wbase + off, _CHUNK)], sems)

    start(0, head0, tail0, rel0, sem0)

    def body(j, carry):
        even = 2 * j
        start(even + 1, head1, tail1, rel1, sem1)
        wait_chunk(even, head0, tail0, rel0, sem0)
        compute(even, head0, tail0, rel0, score0, sems0)
        start(even + 2, head0, tail0, rel0, sem0)
        wait_chunk(even + 1, head1, tail1, rel1, sem1)
        compute(even + 1, head1, tail1, rel1, score1, sems1)
        return carry

    lax.fori_loop(0, _NPAIR, body, 0)
    wait_chunk(_NCHUNK - 1, head0, tail0, rel0, sem0)
    compute(_NCHUNK - 1, head0, tail0, rel0, score0, sems0)
    pltpu.make_async_copy(
        score1, out_hbm.at[pl.ds(wbase + (_NCHUNK - 2) * _CHUNK, _CHUNK)],
        sems1).wait()
    pltpu.make_async_copy(
        score0, out_hbm.at[pl.ds(wbase + (_NCHUNK - 1) * _CHUNK, _CHUNK)],
        sems0).wait()


def kernel(node_emb, edge_index, rel_emb):
    src = edge_index[0].astype(jnp.int32).reshape(_NW, _NCHUNK, _CHUNK)
    dst = edge_index[1].astype(jnp.int32).reshape(_NW, _NCHUNK, _CHUNK)
    node_perm = node_emb.reshape(_N_NODES, 4, 2, 16).transpose(0, 1, 3, 2)
    node_bf = node_perm.reshape(_N_NODES, _D).astype(jnp.bfloat16)
    node_packed = jax.lax.bitcast_convert_type(
        node_bf.reshape(_N_NODES, _DW, 2), jnp.float32)

    mesh = plsc.VectorSubcoreMesh(core_axis_name="c", subcore_axis_name="s")
    f = pl.kernel(
        _sc_body,
        out_type=jax.ShapeDtypeStruct((_N_EDGES,), jnp.float32),
        mesh=mesh,
        compiler_params=pltpu.CompilerParams(needs_layout_passes=False,
                                             use_tc_tiling_on_sc=False),
        scratch_types=[
            pltpu.VMEM((_NCHUNK, _CHUNK), jnp.int32),
            pltpu.VMEM((_NCHUNK, _CHUNK), jnp.int32),
            pltpu.VMEM((_CHUNK, _DW), jnp.float32),
            pltpu.VMEM((_CHUNK, _DW), jnp.float32),
            pltpu.VMEM((_CHUNK, _D), jnp.float32),
            pltpu.VMEM((_CHUNK,), jnp.float32),
            pltpu.VMEM((_CHUNK, _DW), jnp.float32),
            pltpu.VMEM((_CHUNK, _DW), jnp.float32),
            pltpu.VMEM((_CHUNK, _D), jnp.float32),
            pltpu.VMEM((_CHUNK,), jnp.float32),
            pltpu.VMEM((16, 17), jnp.float32),
            pltpu.SemaphoreType.DMA,
            pltpu.SemaphoreType.DMA,
            pltpu.SemaphoreType.DMA,
            pltpu.SemaphoreType.DMA,
        ],
    )
    return f(node_packed, src, dst, rel_emb)
